# Initial kernel scaffold; baseline (speedup 1.0000x reference)
import jax
import jax.numpy as jnp
from jax.experimental import pallas as pl


def kernel(input_word, input_char, input_pos, input_bert, word_table, char_table, pos_table, conv_w, conv_b, position_table):
    raise NotImplementedError("write your pallas kernel here")


# speedup vs baseline: 2.8174x; 2.8174x over previous
import functools

import jax
import jax.numpy as jnp
from jax import lax
from jax.experimental import pallas as pl
from jax.experimental.pallas import tpu as pltpu
from jax.experimental.pallas import tpu_sc as plsc

B, L, W = 1024, 50, 16
T = B * L
WORD_DIM, CHAR_DIM, POS_DIM, PE_DIM = 128, 32, 32, 50
OUT_DIM = WORD_DIM + 32 + POS_DIM + PE_DIM
NFILT, KSZ = 32, 3
WP = 24
NVALID = W + 4 - KSZ + 1

NC, NS = 2, 16
NW = NC * NS
ROWS_PER_W = T // NW
CHUNK = 800
STREAM = 100
N_STREAM = ROWS_PER_W // STREAM

BLK = 512
GRID = T // BLK
RB = BLK * WP


def _sc_word_gather(word_table, idx2d):
    mesh = plsc.VectorSubcoreMesh(core_axis_name="c", subcore_axis_name="s")

    @functools.partial(
        pl.kernel,
        out_type=jax.ShapeDtypeStruct((T, WORD_DIM), jnp.float32),
        mesh=mesh,
        scratch_types=[
            pltpu.VMEM((N_STREAM, STREAM), jnp.int32),
            pltpu.VMEM((CHUNK, WORD_DIM), jnp.float32),
            pltpu.SemaphoreType.DMA,
        ],
    )
    def k(table_hbm, idx_hbm, out_hbm, idx_v, rows_v, sem):
        wid = lax.axis_index("s") * NC + lax.axis_index("c")
        base = wid * ROWS_PER_W
        pltpu.sync_copy(idx_hbm.at[pl.ds(wid * N_STREAM, N_STREAM)], idx_v)
        streams_per_chunk = CHUNK // STREAM
        for c in range(ROWS_PER_W // CHUNK):
            descs = []
            for j in range(streams_per_chunk):
                d = pltpu.async_copy(
                    table_hbm.at[idx_v.at[c * streams_per_chunk + j]],
                    rows_v.at[pl.ds(j * STREAM, STREAM)],
                    sem,
                )
                descs.append(d)
            for d in descs:
                d.wait()
            pltpu.sync_copy(rows_v, out_hbm.at[pl.ds(base + c * CHUNK, CHUNK)])

    return k(word_table, idx2d)


def _tc_assemble(word, cp3, pos3, char_table, pos_table, wt96, b2, pe50):
    def body(word_ref, cp_ref, pos_ref, ct_ref, pt_ref, w_ref, b_ref, pe_ref, out_ref):
        i = pl.program_id(0)
        cp = cp_ref[0, 0, :]
        oh = (cp[:, None] == lax.broadcasted_iota(jnp.int32, (RB, 256), 1)).astype(jnp.float32)
        emb = jnp.dot(oh, ct_ref[...], preferred_element_type=jnp.float32)
        g0 = jnp.dot(emb, w_ref[0:32, :], preferred_element_type=jnp.float32)
        g1 = jnp.dot(emb, w_ref[32:64, :], preferred_element_type=jnp.float32)
        g2 = jnp.dot(emb, w_ref[64:96, :], preferred_element_type=jnp.float32)
        h = g0
        h = h + jnp.pad(g1[1:, :], ((0, 1), (0, 0)))
        h = h + jnp.pad(g2[2:, :], ((0, 2), (0, 0)))
        rp = lax.broadcasted_iota(jnp.int32, (RB, NFILT), 0) % WP
        hm = jnp.where(rp >= NVALID, -1e30, h)
        m = jnp.max(hm.reshape(BLK, WP, NFILT), axis=1)
        ch = jnp.tanh(m + b_ref[...])
        pidx = pos_ref[0, 0, :]
        ohp = (pidx[:, None] == lax.broadcasted_iota(jnp.int32, (BLK, 64), 1)).astype(jnp.float32)
        pos = jnp.dot(ohp, pt_ref[...], preferred_element_type=jnp.float32)
        li = (lax.broadcasted_iota(jnp.int32, (BLK, PE_DIM), 0) + i * BLK) % L
        ohl = (li == lax.broadcasted_iota(jnp.int32, (BLK, PE_DIM), 1)).astype(jnp.float32)
        pe = jnp.dot(ohl, pe_ref[...], preferred_element_type=jnp.float32)
        out_ref[:, 0:WORD_DIM] = word_ref[...]
        out_ref[:, WORD_DIM:WORD_DIM + 32] = ch
        out_ref[:, WORD_DIM + 32:WORD_DIM + 64] = pos
        out_ref[:, WORD_DIM + 64:OUT_DIM] = pe

    return pl.pallas_call(
        body,
        grid=(GRID,),
        in_specs=[
            pl.BlockSpec((BLK, WORD_DIM), lambda i: (i, 0)),
            pl.BlockSpec((1, 1, RB), lambda i: (i, 0, 0)),
            pl.BlockSpec((1, 1, BLK), lambda i: (i, 0, 0)),
            pl.BlockSpec((256, 32), lambda i: (0, 0)),
            pl.BlockSpec((64, 32), lambda i: (0, 0)),
            pl.BlockSpec((96, 32), lambda i: (0, 0)),
            pl.BlockSpec((1, 32), lambda i: (0, 0)),
            pl.BlockSpec((PE_DIM, PE_DIM), lambda i: (0, 0)),
        ],
        out_specs=pl.BlockSpec((BLK, OUT_DIM), lambda i: (i, 0)),
        out_shape=jax.ShapeDtypeStruct((T, OUT_DIM), jnp.float32),
    )(word, cp3, pos3, char_table, pos_table, wt96, b2, pe50)


def kernel(input_word, input_char, input_pos, input_bert, word_table, char_table, pos_table, conv_w, conv_b, position_table):
    del input_bert
    i32 = jnp.int32
    idx2d = input_word.astype(i32).reshape(T // STREAM, STREAM)
    chars = input_char.astype(i32).reshape(T, W)
    cp = jnp.concatenate(
        [jnp.full((T, 2), -1, i32), chars, jnp.full((T, WP - W - 2), -1, i32)], axis=1
    ).reshape(GRID, 1, RB)
    pos3 = input_pos.astype(i32).reshape(GRID, 1, BLK)
    wt96 = conv_w.transpose(2, 1, 0).reshape(KSZ * 32, NFILT)
    b2 = conv_b.reshape(1, NFILT)
    pe50 = position_table[:L]
    word = _sc_word_gather(word_table, idx2d)
    out = _tc_assemble(word, cp, pos3, char_table, pos_table, wt96, b2, pe50)
    return out.reshape(B, L, OUT_DIM)


# speedup vs baseline: 5.4059x; 1.9188x over previous
import functools

import jax
import jax.numpy as jnp
from jax import lax
from jax.experimental import pallas as pl
from jax.experimental.pallas import tpu as pltpu
from jax.experimental.pallas import tpu_sc as plsc

B, L, W = 1024, 50, 16
T = B * L
WORD_DIM, CHAR_DIM, POS_DIM, PE_DIM = 128, 32, 32, 50
OUT_DIM = WORD_DIM + 32 + POS_DIM + PE_DIM
NFILT, KSZ = 32, 3
WP = 24
NVALID = W + 4 - KSZ + 1

NC, NS = 2, 16
NW = NC * NS
ROWS_PER_W = T // NW
CHUNK = 800
STREAM = 100
N_STREAM = ROWS_PER_W // STREAM

BLK = 512
GRID = T // BLK
RB = BLK * W


def _sc_word_gather(word_table, idx2d):
    mesh = plsc.VectorSubcoreMesh(core_axis_name="c", subcore_axis_name="s")

    @functools.partial(
        pl.kernel,
        out_type=jax.ShapeDtypeStruct((T, WORD_DIM), jnp.float32),
        mesh=mesh,
        scratch_types=[
            pltpu.VMEM((N_STREAM, STREAM), jnp.int32),
            pltpu.VMEM((CHUNK, WORD_DIM), jnp.float32),
            pltpu.SemaphoreType.DMA,
        ],
    )
    def k(table_hbm, idx_hbm, out_hbm, idx_v, rows_v, sem):
        wid = lax.axis_index("s") * NC + lax.axis_index("c")
        base = wid * ROWS_PER_W
        pltpu.sync_copy(idx_hbm.at[pl.ds(wid * N_STREAM, N_STREAM)], idx_v)
        streams_per_chunk = CHUNK // STREAM
        for c in range(ROWS_PER_W // CHUNK):
            descs = []
            for j in range(streams_per_chunk):
                d = pltpu.async_copy(
                    table_hbm.at[idx_v.at[c * streams_per_chunk + j]],
                    rows_v.at[pl.ds(j * STREAM, STREAM)],
                    sem,
                )
                descs.append(d)
            for d in descs:
                d.wait()
            pltpu.sync_copy(rows_v, out_hbm.at[pl.ds(base + c * CHUNK, CHUNK)])

    return k(word_table, idx2d)


def _tc_assemble(word, cp3, pos3, tw_bf16, b2, pos_table, pe50):
    def body(word_ref, cp_ref, pos_ref, tw_ref, b_ref, pt_ref, pe_ref, out_ref, *g_ref):
        i = pl.program_id(0)
        cp = cp_ref[0, 0, :]
        oh = (cp[:, None] == lax.broadcasted_iota(jnp.int32, (RB, 256), 1)).astype(jnp.bfloat16)
        for k in range(KSZ):
            g_ref[k][...] = jnp.dot(
                oh, tw_ref[pl.ds(256 * k, 256), :], preferred_element_type=jnp.float32
            )
        m = None
        for t in range(NVALID):
            acc = None
            for k in range(KSZ):
                w0 = t + k - 2
                if 0 <= w0 < W:
                    sl = g_ref[k][pl.ds(BLK * w0, BLK), :]
                    acc = sl if acc is None else acc + sl
            m = acc if m is None else jnp.maximum(m, acc)
        ch = jnp.tanh(m + b_ref[...])
        pidx = pos_ref[0, 0, :]
        ohp = (pidx[:, None] == lax.broadcasted_iota(jnp.int32, (BLK, 64), 1)).astype(jnp.float32)
        pos = jnp.dot(ohp, pt_ref[...], preferred_element_type=jnp.float32)
        li = (lax.broadcasted_iota(jnp.int32, (BLK, PE_DIM), 0) + i * BLK) % L
        ohl = (li == lax.broadcasted_iota(jnp.int32, (BLK, PE_DIM), 1)).astype(jnp.float32)
        pe = jnp.dot(ohl, pe_ref[...], preferred_element_type=jnp.float32)
        out_ref[:, 0:WORD_DIM] = word_ref[...]
        out_ref[:, WORD_DIM:WORD_DIM + 32] = ch
        out_ref[:, WORD_DIM + 32:WORD_DIM + 64] = pos
        out_ref[:, WORD_DIM + 64:OUT_DIM] = pe

    return pl.pallas_call(
        body,
        grid=(GRID,),
        in_specs=[
            pl.BlockSpec((BLK, WORD_DIM), lambda i: (i, 0)),
            pl.BlockSpec((1, 1, RB), lambda i: (i, 0, 0)),
            pl.BlockSpec((1, 1, BLK), lambda i: (i, 0, 0)),
            pl.BlockSpec((KSZ * 256, NFILT), lambda i: (0, 0)),
            pl.BlockSpec((1, 32), lambda i: (0, 0)),
            pl.BlockSpec((64, 32), lambda i: (0, 0)),
            pl.BlockSpec((PE_DIM, PE_DIM), lambda i: (0, 0)),
        ],
        out_specs=pl.BlockSpec((BLK, OUT_DIM), lambda i: (i, 0)),
        out_shape=jax.ShapeDtypeStruct((T, OUT_DIM), jnp.float32),
        scratch_shapes=[pltpu.VMEM((RB, NFILT), jnp.float32)] * KSZ,
    )(word, cp3, pos3, tw_bf16, b2, pos_table, pe50)


def kernel(input_word, input_char, input_pos, input_bert, word_table, char_table, pos_table, conv_w, conv_b, position_table):
    del input_bert
    i32 = jnp.int32
    idx2d = input_word.astype(i32).reshape(T // STREAM, STREAM)
    cp3 = input_char.astype(i32).reshape(GRID, BLK, W).transpose(0, 2, 1).reshape(GRID, 1, RB)
    pos3 = input_pos.astype(i32).reshape(GRID, 1, BLK)
    tw = jnp.concatenate([char_table @ conv_w[:, :, k].T for k in range(KSZ)], axis=0)
    tw_bf16 = tw.astype(jnp.bfloat16)
    b2 = conv_b.reshape(1, NFILT)
    pe50 = position_table[:L]
    word = _sc_word_gather(word_table, idx2d)
    out = _tc_assemble(word, cp3, pos3, tw_bf16, b2, pos_table, pe50)
    return out.reshape(B, L, OUT_DIM)


# speedup vs baseline: 5.4464x; 1.0075x over previous
import functools

import jax
import jax.numpy as jnp
from jax import lax
from jax.experimental import pallas as pl
from jax.experimental.pallas import tpu as pltpu
from jax.experimental.pallas import tpu_sc as plsc

B, L, W = 1024, 50, 16
T = B * L
WORD_DIM, CHAR_DIM, POS_DIM, PE_DIM = 128, 32, 32, 50
OUT_DIM = WORD_DIM + 32 + POS_DIM + PE_DIM
NFILT, KSZ = 32, 3
WP = 24
NVALID = W + 4 - KSZ + 1

NC, NS = 2, 16
NW = NC * NS
ROWS_PER_W = T // NW
CHUNK = 800
STREAM = 100
N_STREAM = ROWS_PER_W // STREAM

BLK = 1024
GRID = T // BLK
RB = BLK * W
SPAD = RB + 2 * BLK


def _sc_word_gather(word_table, idx2d):
    mesh = plsc.VectorSubcoreMesh(core_axis_name="c", subcore_axis_name="s")

    @functools.partial(
        pl.kernel,
        out_type=jax.ShapeDtypeStruct((T, WORD_DIM), jnp.float32),
        mesh=mesh,
        scratch_types=[
            pltpu.VMEM((N_STREAM, STREAM), jnp.int32),
            pltpu.VMEM((CHUNK, WORD_DIM), jnp.float32),
            pltpu.SemaphoreType.DMA,
        ],
    )
    def k(table_hbm, idx_hbm, out_hbm, idx_v, rows_v, sem):
        wid = lax.axis_index("s") * NC + lax.axis_index("c")
        base = wid * ROWS_PER_W
        pltpu.sync_copy(idx_hbm.at[pl.ds(wid * N_STREAM, N_STREAM)], idx_v)
        streams_per_chunk = CHUNK // STREAM
        for c in range(ROWS_PER_W // CHUNK):
            descs = []
            for j in range(streams_per_chunk):
                d = pltpu.async_copy(
                    table_hbm.at[idx_v.at[c * streams_per_chunk + j]],
                    rows_v.at[pl.ds(j * STREAM, STREAM)],
                    sem,
                )
                descs.append(d)
            for d in descs:
                d.wait()
            pltpu.sync_copy(rows_v, out_hbm.at[pl.ds(base + c * CHUNK, CHUNK)])

    return k(word_table, idx2d)


def _tc_assemble(word, cp3, pos3, tw_bf16, b2, pos_table, pe50):
    def body(word_ref, cp_ref, pos_ref, tw_ref, b_ref, pt_ref, pe_ref, out_ref, *g_ref):
        i = pl.program_id(0)
        cp = cp_ref[0, 0, :]
        oh = (cp[:, None] == lax.broadcasted_iota(jnp.int32, (RB, 256), 1)).astype(jnp.bfloat16)
        for k in range(KSZ):
            g_ref[k][...] = jnp.dot(
                oh, tw_ref[pl.ds(256 * k, 256), :], preferred_element_type=jnp.float32
            )
        m = None
        for t in range(NVALID):
            acc = None
            for k in range(KSZ):
                w0 = t + k - 2
                if 0 <= w0 < W:
                    sl = g_ref[k][pl.ds(BLK * w0, BLK), :]
                    acc = sl if acc is None else acc + sl
            m = acc if m is None else jnp.maximum(m, acc)
        ch = jnp.tanh(m + b_ref[...])
        pidx = pos_ref[0, 0, :]
        ohp = (pidx[:, None] == lax.broadcasted_iota(jnp.int32, (BLK, 64), 1)).astype(jnp.float32)
        pos = jnp.dot(ohp, pt_ref[...], preferred_element_type=jnp.float32)
        li = (lax.broadcasted_iota(jnp.int32, (BLK, PE_DIM), 0) + i * BLK) % L
        ohl = (li == lax.broadcasted_iota(jnp.int32, (BLK, PE_DIM), 1)).astype(jnp.float32)
        pe = jnp.dot(ohl, pe_ref[...], preferred_element_type=jnp.float32)
        out_ref[:, 0:WORD_DIM] = word_ref[...]
        out_ref[:, WORD_DIM:WORD_DIM + 32] = ch
        out_ref[:, WORD_DIM + 32:WORD_DIM + 64] = pos
        out_ref[:, WORD_DIM + 64:OUT_DIM] = pe

    return pl.pallas_call(
        body,
        grid=(GRID,),
        in_specs=[
            pl.BlockSpec((BLK, WORD_DIM), lambda i: (i, 0)),
            pl.BlockSpec((1, 1, RB), lambda i: (i, 0, 0)),
            pl.BlockSpec((1, 1, BLK), lambda i: (i, 0, 0)),
            pl.BlockSpec((KSZ * 256, NFILT), lambda i: (0, 0)),
            pl.BlockSpec((1, 32), lambda i: (0, 0)),
            pl.BlockSpec((64, 32), lambda i: (0, 0)),
            pl.BlockSpec((PE_DIM, PE_DIM), lambda i: (0, 0)),
        ],
        out_specs=pl.BlockSpec((BLK, OUT_DIM), lambda i: (i, 0)),
        out_shape=jax.ShapeDtypeStruct((T, OUT_DIM), jnp.float32),
        scratch_shapes=[pltpu.VMEM((RB, NFILT), jnp.float32)] * KSZ,
    )(word, cp3, pos3, tw_bf16, b2, pos_table, pe50)


def kernel(input_word, input_char, input_pos, input_bert, word_table, char_table, pos_table, conv_w, conv_b, position_table):
    del input_bert
    i32 = jnp.int32
    idx2d = input_word.astype(i32).reshape(T // STREAM, STREAM)
    cp3 = input_char.astype(i32).reshape(GRID, BLK, W).transpose(0, 2, 1).reshape(GRID, 1, RB)
    pos3 = input_pos.astype(i32).reshape(GRID, 1, BLK)
    tw = jnp.concatenate([char_table @ conv_w[:, :, k].T for k in range(KSZ)], axis=0)
    tw_bf16 = tw.astype(jnp.bfloat16)
    b2 = conv_b.reshape(1, NFILT)
    pe50 = position_table[:L]
    word = _sc_word_gather(word_table, idx2d)
    out = _tc_assemble(word, cp3, pos3, tw_bf16, b2, pos_table, pe50)
    return out.reshape(B, L, OUT_DIM)


# speedup vs baseline: 7.7704x; 1.4267x over previous
import functools

import jax
import jax.numpy as jnp
from jax import lax
from jax.experimental import pallas as pl
from jax.experimental.pallas import tpu as pltpu
from jax.experimental.pallas import tpu_sc as plsc

B, L, W = 1024, 50, 16
T = B * L
WORD_DIM, CHAR_DIM, POS_DIM, PE_DIM = 128, 32, 32, 50
OUT_DIM = WORD_DIM + 32 + POS_DIM + PE_DIM
NFILT, KSZ = 32, 3
WP = 24
NVALID = W + 4 - KSZ + 1

NC, NS = 2, 16
NW = NC * NS
ROWS_PER_W = T // NW
CHUNK = 800
STREAM = 100
N_STREAM = ROWS_PER_W // STREAM

BLK = 1024
GRID = T // BLK
RB = BLK * W
SPAD = RB + 2 * BLK


def _sc_word_gather(word_table, idx2d):
    mesh = plsc.VectorSubcoreMesh(core_axis_name="c", subcore_axis_name="s")

    @functools.partial(
        pl.kernel,
        out_type=jax.ShapeDtypeStruct((T, WORD_DIM), jnp.float32),
        mesh=mesh,
        scratch_types=[
            pltpu.VMEM((N_STREAM, STREAM), jnp.int32),
            pltpu.VMEM((CHUNK, WORD_DIM), jnp.float32),
            pltpu.SemaphoreType.DMA,
        ],
    )
    def k(table_hbm, idx_hbm, out_hbm, idx_v, rows_v, sem):
        wid = lax.axis_index("s") * NC + lax.axis_index("c")
        base = wid * ROWS_PER_W
        pltpu.sync_copy(idx_hbm.at[pl.ds(wid * N_STREAM, N_STREAM)], idx_v)
        streams_per_chunk = CHUNK // STREAM
        for c in range(ROWS_PER_W // CHUNK):
            descs = []
            for j in range(streams_per_chunk):
                d = pltpu.async_copy(
                    table_hbm.at[idx_v.at[c * streams_per_chunk + j]],
                    rows_v.at[pl.ds(j * STREAM, STREAM)],
                    sem,
                )
                descs.append(d)
            for d in descs:
                d.wait()
            pltpu.sync_copy(rows_v, out_hbm.at[pl.ds(base + c * CHUNK, CHUNK)])

    return k(word_table, idx2d)


def _tc_assemble(word, cp3, pos3, tw_bf16, b2, pos_table, pe50):
    def body(word_ref, cp_ref, pos_ref, tw_ref, b_ref, pt_ref, pe_ref, out_ref):
        i = pl.program_id(0)
        cp = cp_ref[0, 0, :]
        idxs, shs = [], []
        for w in range(W):
            cpw = cp[BLK * w:BLK * (w + 1)]
            idxs.append(jnp.broadcast_to((cpw & 127)[None, :], (NFILT, BLK)))
            shs.append(jnp.broadcast_to(
                jnp.where(cpw < 128, jnp.int32(16), jnp.int32(0))[None, :], (NFILT, BLK)))

        def garm(k, w0):
            w32 = jnp.take_along_axis(tw_ref[pl.ds(NFILT * k, NFILT), :], idxs[w0], axis=1)
            return lax.bitcast_convert_type((w32 << shs[w0]) & jnp.int32(-65536), jnp.float32)

        mt = None
        for t in range(NVALID):
            acc = None
            for k in range(KSZ):
                w0 = t + k - 2
                if 0 <= w0 < W:
                    sl = garm(k, w0)
                    acc = sl if acc is None else acc + sl
            mt = acc if mt is None else jnp.maximum(mt, acc)
        ch = jnp.tanh(mt.T + b_ref[...])
        pidx = pos_ref[0, 0, :]
        ohp = (pidx[:, None] == lax.broadcasted_iota(jnp.int32, (BLK, 64), 1)).astype(jnp.float32)
        pos = jnp.dot(ohp, pt_ref[...], preferred_element_type=jnp.float32)
        li = (lax.broadcasted_iota(jnp.int32, (BLK, PE_DIM), 0) + i * BLK) % L
        ohl = (li == lax.broadcasted_iota(jnp.int32, (BLK, PE_DIM), 1)).astype(jnp.float32)
        pe = jnp.dot(ohl, pe_ref[...], preferred_element_type=jnp.float32)
        out_ref[:, 0:WORD_DIM] = word_ref[...]
        out_ref[:, WORD_DIM:WORD_DIM + 32] = ch
        out_ref[:, WORD_DIM + 32:WORD_DIM + 64] = pos
        out_ref[:, WORD_DIM + 64:OUT_DIM] = pe

    return pl.pallas_call(
        body,
        grid=(GRID,),
        in_specs=[
            pl.BlockSpec((BLK, WORD_DIM), lambda i: (i, 0)),
            pl.BlockSpec((1, 1, RB), lambda i: (i, 0, 0)),
            pl.BlockSpec((1, 1, BLK), lambda i: (i, 0, 0)),
            pl.BlockSpec((KSZ * NFILT, 128), lambda i: (0, 0)),
            pl.BlockSpec((1, 32), lambda i: (0, 0)),
            pl.BlockSpec((64, 32), lambda i: (0, 0)),
            pl.BlockSpec((PE_DIM, PE_DIM), lambda i: (0, 0)),
        ],
        out_specs=pl.BlockSpec((BLK, OUT_DIM), lambda i: (i, 0)),
        out_shape=jax.ShapeDtypeStruct((T, OUT_DIM), jnp.float32),
    )(word, cp3, pos3, tw_bf16, b2, pos_table, pe50)


def kernel(input_word, input_char, input_pos, input_bert, word_table, char_table, pos_table, conv_w, conv_b, position_table):
    del input_bert
    i32 = jnp.int32
    idx2d = input_word.astype(i32).reshape(T // STREAM, STREAM)
    cp3 = input_char.astype(i32).reshape(GRID, BLK, W).transpose(0, 2, 1).reshape(GRID, 1, RB)
    pos3 = input_pos.astype(i32).reshape(GRID, 1, BLK)
    tw = jnp.concatenate([conv_w[:, :, k] @ char_table.T for k in range(KSZ)], axis=0)
    tlo = tw[:, :128].astype(jnp.bfloat16).view(jnp.uint16).astype(jnp.uint32)
    thi = tw[:, 128:].astype(jnp.bfloat16).view(jnp.uint16).astype(jnp.uint32)
    tw = ((thi << 16) | tlo).astype(jnp.int32)
    b2 = conv_b.reshape(1, NFILT)
    pe50 = position_table[:L]
    word = _sc_word_gather(word_table, idx2d)
    out = _tc_assemble(word, cp3, pos3, tw, b2, pos_table, pe50)
    return out.reshape(B, L, OUT_DIM)
